# Initial kernel scaffold; baseline (speedup 1.0000x reference)
import functools
import math

import jax
import jax.numpy as jnp
from jax import lax
from jax.experimental import pallas as pl
from jax.experimental.pallas import tpu as pltpu
from jax.experimental.pallas import tpu_sc as plsc

N_NODES = 10000
N_EDGES = 320000
HID = 128
N_LAYERS = 4
N_GRAPHS = 128
EMB_DIM = 256
INV_STD = 1.0 / math.sqrt(1.0 + 1e-5)

NC = 2
NS = 16
NW = NC * NS

IDXROWS_PER_TILE = 80
E_PAD = NW * IDXROWS_PER_TILE * 128
CHUNK_IDXROWS = 4
N_CHUNKS = IDXROWS_PER_TILE // CHUNK_IDXROWS
CHUNK_E = CHUNK_IDXROWS * 128
NPAD = 10016
ZROWS_PER_TILE = NPAD // NS
OUT_ROWS_PER_TILE = N_NODES // NS


def _agg_body(h_hbm, src_hbm, dst_hbm, out0_hbm, out1_hbm,
              shared, rows, idxs, idxd, sem):
    cid = lax.axis_index("c")
    sid = lax.axis_index("s")
    wid = sid * NC + cid

    def _zr(r, carry):
        for c in range(HID // 16):
            rows[r, pl.ds(c * 16, 16)] = jnp.zeros((16,), jnp.float32)
        return carry
    lax.fori_loop(0, CHUNK_E, _zr, 0)
    zbase = sid * ZROWS_PER_TILE
    pltpu.sync_copy(rows, shared.at[pl.ds(zbase, CHUNK_E)])
    pltpu.sync_copy(rows.at[pl.ds(0, ZROWS_PER_TILE - CHUNK_E)],
                    shared.at[pl.ds(zbase + CHUNK_E, ZROWS_PER_TILE - CHUNK_E)])
    plsc.subcore_barrier()

    base_row = wid * IDXROWS_PER_TILE

    def _chunk(g, carry):
        r0 = base_row + g * CHUNK_IDXROWS
        pltpu.sync_copy(src_hbm.at[pl.ds(r0, CHUNK_IDXROWS)], idxs)
        pltpu.sync_copy(dst_hbm.at[pl.ds(r0, CHUNK_IDXROWS)], idxd)
        cps = [pltpu.async_copy(h_hbm.at[idxs.at[j]],
                                rows.at[pl.ds(j * 128, 128)], sem)
               for j in range(CHUNK_IDXROWS)]
        for cp in cps:
            cp.wait()
        for j in range(CHUNK_IDXROWS):
            pltpu.sync_copy(rows.at[pl.ds(j * 128, 128)],
                            shared.at[idxd.at[j]], add=True)
        return carry
    lax.fori_loop(0, N_CHUNKS, _chunk, 0)
    plsc.subcore_barrier()

    obase = sid * OUT_ROWS_PER_TILE
    src_slice = shared.at[pl.ds(obase, OUT_ROWS_PER_TILE)]

    @pl.when(cid == 0)
    def _():
        pltpu.sync_copy(src_slice, out0_hbm.at[pl.ds(obase, OUT_ROWS_PER_TILE)])

    @pl.when(cid == 1)
    def _():
        pltpu.sync_copy(src_slice, out1_hbm.at[pl.ds(obase, OUT_ROWS_PER_TILE)])


_agg_kernel = functools.partial(
    pl.kernel,
    out_type=(jax.ShapeDtypeStruct((N_NODES, HID), jnp.float32),
              jax.ShapeDtypeStruct((N_NODES, HID), jnp.float32)),
    mesh=plsc.VectorSubcoreMesh(core_axis_name="c", subcore_axis_name="s",
                                num_cores=NC, num_subcores=NS),
    scratch_types=[
        pltpu.VMEM_SHARED((NPAD, HID), jnp.float32),
        pltpu.VMEM((CHUNK_E, HID), jnp.float32),
        pltpu.VMEM((CHUNK_IDXROWS, 128), jnp.int32),
        pltpu.VMEM((CHUNK_IDXROWS, 128), jnp.int32),
        pltpu.SemaphoreType.DMA,
    ],
)(_agg_body)


def _encode_body(x_ref, w_ref, b_ref, o_ref):
    o_ref[...] = jnp.dot(x_ref[...], w_ref[...],
                         preferred_element_type=jnp.float32) + b_ref[...]


def _mlp_body(h_ref, a0_ref, a1_ref, w1_ref, b1_ref, w2_ref, b2_ref,
              g_ref, bt_ref, sc_ref, o_ref):
    h = h_ref[...]
    m = sc_ref[0, 0] * h + a0_ref[...] + a1_ref[...]
    t = jnp.dot(m, w1_ref[...], preferred_element_type=jnp.float32) + b1_ref[...]
    t = jnp.maximum(t, 0.0)
    t = jnp.dot(t, w2_ref[...], preferred_element_type=jnp.float32) + b2_ref[...]
    t = g_ref[...] * (t * INV_STD) + bt_ref[...]
    o_ref[...] = jnp.maximum(t, 0.0) + h


def _head_body(h_ref, b_ref, wp1_ref, bp1_ref, wp2_ref, bp2_ref, o_ref,
               accp_ref, accc_ref):
    i = pl.program_id(0)

    @pl.when(i == 0)
    def _():
        accp_ref[...] = jnp.zeros_like(accp_ref)
        accc_ref[...] = jnp.zeros_like(accc_ref)

    bb = b_ref[...]
    gi = lax.broadcasted_iota(jnp.float32, (1, N_GRAPHS), 1)
    onehot = (bb == gi).astype(jnp.float32)
    hb = h_ref[...]
    accp_ref[...] += lax.dot_general(
        onehot, hb, (((0,), (0,)), ((), ())),
        preferred_element_type=jnp.float32)
    ones = jnp.ones((bb.shape[0], 1), jnp.float32)
    accc_ref[...] += lax.dot_general(
        onehot, ones, (((0,), (0,)), ((), ())),
        preferred_element_type=jnp.float32)

    @pl.when(i == pl.num_programs(0) - 1)
    def _():
        pooled = accp_ref[...] / jnp.maximum(accc_ref[...], 1.0)
        t = jnp.dot(pooled, wp1_ref[...],
                    preferred_element_type=jnp.float32) + bp1_ref[...]
        t = jnp.maximum(t, 0.0)
        o_ref[...] = jnp.dot(t, wp2_ref[...],
                             preferred_element_type=jnp.float32) + bp2_ref[...]


_BLK = 2000
_HEAD_BLK = 1000


def _full_spec(shape):
    return pl.BlockSpec(shape, lambda i: (0,) * len(shape))


def _encode(x, w, b):
    return pl.pallas_call(
        _encode_body,
        grid=(N_NODES // _BLK,),
        in_specs=[
            pl.BlockSpec((_BLK, HID), lambda i: (i, 0)),
            _full_spec((HID, HID)),
            _full_spec((1, HID)),
        ],
        out_specs=pl.BlockSpec((_BLK, HID), lambda i: (i, 0)),
        out_shape=jax.ShapeDtypeStruct((N_NODES, HID), jnp.float32),
    )(x, w, b)


def _mlp(h, a0, a1, w1, b1, w2, b2, g, bt, sc):
    return pl.pallas_call(
        _mlp_body,
        grid=(N_NODES // _BLK,),
        in_specs=[
            pl.BlockSpec((_BLK, HID), lambda i: (i, 0)),
            pl.BlockSpec((_BLK, HID), lambda i: (i, 0)),
            pl.BlockSpec((_BLK, HID), lambda i: (i, 0)),
            _full_spec((HID, HID)),
            _full_spec((1, HID)),
            _full_spec((HID, HID)),
            _full_spec((1, HID)),
            _full_spec((1, HID)),
            _full_spec((1, HID)),
            pl.BlockSpec(memory_space=pltpu.SMEM),
        ],
        out_specs=pl.BlockSpec((_BLK, HID), lambda i: (i, 0)),
        out_shape=jax.ShapeDtypeStruct((N_NODES, HID), jnp.float32),
    )(h, a0, a1, w1, b1, w2, b2, g, bt, sc)


def _head(h, batchf, wp1, bp1, wp2, bp2):
    return pl.pallas_call(
        _head_body,
        grid=(N_NODES // _HEAD_BLK,),
        in_specs=[
            pl.BlockSpec((_HEAD_BLK, HID), lambda i: (i, 0)),
            pl.BlockSpec((_HEAD_BLK, 1), lambda i: (i, 0)),
            _full_spec((HID, HID)),
            _full_spec((1, HID)),
            _full_spec((HID, EMB_DIM)),
            _full_spec((1, EMB_DIM)),
        ],
        out_specs=_full_spec((N_GRAPHS, EMB_DIM)),
        out_shape=jax.ShapeDtypeStruct((N_GRAPHS, EMB_DIM), jnp.float32),
        scratch_shapes=[
            pltpu.VMEM((N_GRAPHS, HID), jnp.float32),
            pltpu.VMEM((N_GRAPHS, 1), jnp.float32),
        ],
        compiler_params=pltpu.CompilerParams(
            dimension_semantics=("arbitrary",)),
    )(h, batchf, wp1, bp1, wp2, bp2)


def kernel(x, edge_index, batch, W_enc, b_enc, eps, W1, b1, W2, b2,
           gamma, beta, Wp1, bp1, Wp2, bp2):
    src = edge_index[0]
    dst = edge_index[1]
    pad = E_PAD - N_EDGES
    src2 = jnp.concatenate(
        [src, jnp.zeros((pad,), jnp.int32)]).reshape(E_PAD // 128, 128)
    dst2 = jnp.concatenate(
        [dst, jnp.full((pad,), N_NODES, jnp.int32)]).reshape(E_PAD // 128, 128)
    batchf = batch.astype(jnp.float32).reshape(N_NODES, 1)

    h = _encode(x, W_enc, b_enc.reshape(1, HID))
    for i in range(N_LAYERS):
        a0, a1 = _agg_kernel(h, src2, dst2)
        h = _mlp(h, a0, a1,
                 W1[i], b1[i].reshape(1, HID),
                 W2[i], b2[i].reshape(1, HID),
                 gamma[i].reshape(1, HID), beta[i].reshape(1, HID),
                 (1.0 + eps[i]).reshape(1, 1))
    return _head(h, batchf, Wp1, bp1.reshape(1, HID), Wp2, bp2.reshape(1, EMB_DIM))


# speedup vs baseline: 2.9040x; 2.9040x over previous
import functools
import math

import jax
import jax.numpy as jnp
from jax import lax
from jax.experimental import pallas as pl
from jax.experimental.pallas import tpu as pltpu
from jax.experimental.pallas import tpu_sc as plsc

N_NODES = 10000
N_EDGES = 320000
HID = 128
N_LAYERS = 4
N_GRAPHS = 128
EMB_DIM = 256
INV_STD = 1.0 / math.sqrt(1.0 + 1e-5)

NC = 2
NS = 16
NW = NC * NS

IDXROWS_PER_TILE = 80
E_PAD = NW * IDXROWS_PER_TILE * 128
CHUNK_IDXROWS = 8
N_CHUNKS = IDXROWS_PER_TILE // CHUNK_IDXROWS
HALF_IDXROWS = 2
CHUNK_E = HALF_IDXROWS * 128
NPAD = 10016
ZSPLIT = 632
OSPLIT = 632


def _agg_body(h_hbm, src_hbm, dst_hbm, out0_hbm, out1_hbm,
              shared, rows, idxs, idxd, sem):
    cid = lax.axis_index("c")
    sid = lax.axis_index("s")
    wid = sid * NC + cid

    def _zr(r, carry):
        for c in range(HID // 16):
            rows[r, pl.ds(c * 16, 16)] = jnp.zeros((16,), jnp.float32)
        return carry
    lax.fori_loop(0, CHUNK_E, _zr, 0)
    zbase = sid * ZSPLIT

    @pl.when(sid < NS - 1)
    def _():
        for z in range(ZSPLIT // CHUNK_E):
            pltpu.sync_copy(rows, shared.at[pl.ds(zbase + z * CHUNK_E, CHUNK_E)])
        zrem = ZSPLIT % CHUNK_E
        pltpu.sync_copy(rows.at[pl.ds(0, zrem)],
                        shared.at[pl.ds(zbase + ZSPLIT - zrem, zrem)])

    @pl.when(sid == NS - 1)
    def _():
        zlast = NPAD - (NS - 1) * ZSPLIT
        lbase = (NS - 1) * ZSPLIT
        for z in range(zlast // CHUNK_E):
            pltpu.sync_copy(rows, shared.at[pl.ds(lbase + z * CHUNK_E, CHUNK_E)])
        zrem = zlast % CHUNK_E
        pltpu.sync_copy(rows.at[pl.ds(0, zrem)],
                        shared.at[pl.ds(lbase + zlast - zrem, zrem)])
    plsc.subcore_barrier()

    base_row = wid * IDXROWS_PER_TILE

    def _chunk(g, carry):
        r0 = base_row + g * CHUNK_IDXROWS
        pltpu.sync_copy(src_hbm.at[pl.ds(r0, CHUNK_IDXROWS)], idxs)
        pltpu.sync_copy(dst_hbm.at[pl.ds(r0, CHUNK_IDXROWS)], idxd)
        for half in range(CHUNK_IDXROWS // HALF_IDXROWS):
            cps = [pltpu.async_copy(
                       h_hbm.at[idxs.at[half * HALF_IDXROWS + j]],
                       rows.at[pl.ds(j * 128, 128)], sem)
                   for j in range(HALF_IDXROWS)]
            for cp in cps:
                cp.wait()
            for j in range(HALF_IDXROWS):
                pltpu.sync_copy(rows.at[pl.ds(j * 128, 128)],
                                shared.at[idxd.at[half * HALF_IDXROWS + j]],
                                add=True)
        return carry
    lax.fori_loop(0, N_CHUNKS, _chunk, 0)
    plsc.subcore_barrier()

    obase = sid * OSPLIT

    @pl.when(jnp.logical_and(cid == 0, sid < NS - 1))
    def _():
        pltpu.sync_copy(shared.at[pl.ds(obase, OSPLIT)],
                        out0_hbm.at[pl.ds(obase, OSPLIT)])

    @pl.when(jnp.logical_and(cid == 0, sid == NS - 1))
    def _():
        olast = N_NODES - (NS - 1) * OSPLIT
        pltpu.sync_copy(shared.at[pl.ds((NS - 1) * OSPLIT, olast)],
                        out0_hbm.at[pl.ds((NS - 1) * OSPLIT, olast)])

    @pl.when(jnp.logical_and(cid == 1, sid < NS - 1))
    def _():
        pltpu.sync_copy(shared.at[pl.ds(obase, OSPLIT)],
                        out1_hbm.at[pl.ds(obase, OSPLIT)])

    @pl.when(jnp.logical_and(cid == 1, sid == NS - 1))
    def _():
        olast = N_NODES - (NS - 1) * OSPLIT
        pltpu.sync_copy(shared.at[pl.ds((NS - 1) * OSPLIT, olast)],
                        out1_hbm.at[pl.ds((NS - 1) * OSPLIT, olast)])


_agg_kernel = functools.partial(
    pl.kernel,
    out_type=(jax.ShapeDtypeStruct((N_NODES, HID), jnp.float32),
              jax.ShapeDtypeStruct((N_NODES, HID), jnp.float32)),
    mesh=plsc.VectorSubcoreMesh(core_axis_name="c", subcore_axis_name="s",
                                num_cores=NC, num_subcores=NS),
    scratch_types=[
        pltpu.VMEM_SHARED((NPAD, HID), jnp.float32),
        pltpu.VMEM((CHUNK_E, HID), jnp.float32),
        pltpu.VMEM((CHUNK_IDXROWS, 128), jnp.int32),
        pltpu.VMEM((CHUNK_IDXROWS, 128), jnp.int32),
        pltpu.SemaphoreType.DMA,
    ],
)(_agg_body)


def _encode_body(x_ref, w_ref, b_ref, o_ref):
    o_ref[...] = jnp.dot(x_ref[...], w_ref[...],
                         preferred_element_type=jnp.float32) + b_ref[...]


def _mlp_body(h_ref, a0_ref, a1_ref, w1_ref, b1_ref, w2_ref, b2_ref,
              g_ref, bt_ref, sc_ref, o_ref):
    h = h_ref[...]
    m = sc_ref[0, 0] * h + a0_ref[...] + a1_ref[...]
    t = jnp.dot(m, w1_ref[...], preferred_element_type=jnp.float32) + b1_ref[...]
    t = jnp.maximum(t, 0.0)
    t = jnp.dot(t, w2_ref[...], preferred_element_type=jnp.float32) + b2_ref[...]
    t = g_ref[...] * (t * INV_STD) + bt_ref[...]
    o_ref[...] = jnp.maximum(t, 0.0) + h


def _head_body(h_ref, b_ref, wp1_ref, bp1_ref, wp2_ref, bp2_ref, o_ref,
               accp_ref, accc_ref):
    i = pl.program_id(0)

    @pl.when(i == 0)
    def _():
        accp_ref[...] = jnp.zeros_like(accp_ref)
        accc_ref[...] = jnp.zeros_like(accc_ref)

    bb = b_ref[...]
    gi = lax.broadcasted_iota(jnp.int32, (1, N_GRAPHS), 1)
    onehot = (bb == gi).astype(jnp.float32)
    hb = h_ref[...]
    accp_ref[...] += lax.dot_general(
        onehot, hb, (((0,), (0,)), ((), ())),
        preferred_element_type=jnp.float32)
    ones = jnp.ones((bb.shape[0], 1), jnp.float32)
    accc_ref[...] += lax.dot_general(
        onehot, ones, (((0,), (0,)), ((), ())),
        preferred_element_type=jnp.float32)

    @pl.when(i == pl.num_programs(0) - 1)
    def _():
        pooled = accp_ref[...] / jnp.maximum(accc_ref[...], 1.0)
        t = jnp.dot(pooled, wp1_ref[...],
                    preferred_element_type=jnp.float32) + bp1_ref[...]
        t = jnp.maximum(t, 0.0)
        o_ref[...] = jnp.dot(t, wp2_ref[...],
                             preferred_element_type=jnp.float32) + bp2_ref[...]


_BLK = 2000
_HEAD_BLK = 1000


def _full_spec(shape):
    return pl.BlockSpec(shape, lambda i: (0,) * len(shape))


def _encode(x, w, b):
    return pl.pallas_call(
        _encode_body,
        grid=(N_NODES // _BLK,),
        in_specs=[
            pl.BlockSpec((_BLK, HID), lambda i: (i, 0)),
            _full_spec((HID, HID)),
            _full_spec((1, HID)),
        ],
        out_specs=pl.BlockSpec((_BLK, HID), lambda i: (i, 0)),
        out_shape=jax.ShapeDtypeStruct((N_NODES, HID), jnp.float32),
    )(x, w, b)


def _mlp(h, a0, a1, w1, b1, w2, b2, g, bt, sc):
    return pl.pallas_call(
        _mlp_body,
        grid=(N_NODES // _BLK,),
        in_specs=[
            pl.BlockSpec((_BLK, HID), lambda i: (i, 0)),
            pl.BlockSpec((_BLK, HID), lambda i: (i, 0)),
            pl.BlockSpec((_BLK, HID), lambda i: (i, 0)),
            _full_spec((HID, HID)),
            _full_spec((1, HID)),
            _full_spec((HID, HID)),
            _full_spec((1, HID)),
            _full_spec((1, HID)),
            _full_spec((1, HID)),
            pl.BlockSpec(memory_space=pltpu.SMEM),
        ],
        out_specs=pl.BlockSpec((_BLK, HID), lambda i: (i, 0)),
        out_shape=jax.ShapeDtypeStruct((N_NODES, HID), jnp.float32),
    )(h, a0, a1, w1, b1, w2, b2, g, bt, sc)


def _head(h, batchf, wp1, bp1, wp2, bp2):
    return pl.pallas_call(
        _head_body,
        grid=(N_NODES // _HEAD_BLK,),
        in_specs=[
            pl.BlockSpec((_HEAD_BLK, HID), lambda i: (i, 0)),
            pl.BlockSpec((_HEAD_BLK, 1), lambda i: (i, 0)),
            _full_spec((HID, HID)),
            _full_spec((1, HID)),
            _full_spec((HID, EMB_DIM)),
            _full_spec((1, EMB_DIM)),
        ],
        out_specs=_full_spec((N_GRAPHS, EMB_DIM)),
        out_shape=jax.ShapeDtypeStruct((N_GRAPHS, EMB_DIM), jnp.float32),
        scratch_shapes=[
            pltpu.VMEM((N_GRAPHS, HID), jnp.float32),
            pltpu.VMEM((N_GRAPHS, 1), jnp.float32),
        ],
        compiler_params=pltpu.CompilerParams(
            dimension_semantics=("arbitrary",)),
    )(h, batchf, wp1, bp1, wp2, bp2)


def kernel(x, edge_index, batch, W_enc, b_enc, eps, W1, b1, W2, b2,
           gamma, beta, Wp1, bp1, Wp2, bp2):
    src = edge_index[0]
    dst = edge_index[1]
    pad = E_PAD - N_EDGES
    src2 = jnp.concatenate(
        [src, jnp.zeros((pad,), jnp.int32)]).reshape(E_PAD // 128, 128)
    dst2 = jnp.concatenate(
        [dst, jnp.full((pad,), N_NODES, jnp.int32)]).reshape(E_PAD // 128, 128)
    batchf = batch.reshape(N_NODES, 1)

    h = _encode(x, W_enc, b_enc.reshape(1, HID))
    for i in range(N_LAYERS):
        a0, a1 = _agg_kernel(h, src2, dst2)
        h = _mlp(h, a0, a1,
                 W1[i], b1[i].reshape(1, HID),
                 W2[i], b2[i].reshape(1, HID),
                 gamma[i].reshape(1, HID), beta[i].reshape(1, HID),
                 (1.0 + eps[i]).reshape(1, 1))
    return _head(h, batchf, Wp1, bp1.reshape(1, HID), Wp2, bp2.reshape(1, EMB_DIM))


# speedup vs baseline: 3.2605x; 1.1228x over previous
import functools
import math

import jax
import jax.numpy as jnp
from jax import lax
from jax.experimental import pallas as pl
from jax.experimental.pallas import tpu as pltpu
from jax.experimental.pallas import tpu_sc as plsc

N_NODES = 10000
N_EDGES = 320000
HID = 128
N_LAYERS = 4
N_GRAPHS = 128
EMB_DIM = 256
INV_STD = 1.0 / math.sqrt(1.0 + 1e-5)

NC = 2
NS = 16
NW = NC * NS

IDXROWS_PER_TILE = 80
E_PAD = NW * IDXROWS_PER_TILE * 128
PHASE_IDXROWS = 40
N_PHASES = IDXROWS_PER_TILE // PHASE_IDXROWS
GROWS = 128
NPAD = 10016
ZSPLIT = 632
OSPLIT = 632


def _agg_body(h_hbm, src_hbm, dst_hbm, out0_hbm, out1_hbm,
              shared, rows0, rows1, sidx, didx, sem0, sem1):
    cid = lax.axis_index("c")
    sid = lax.axis_index("s")
    wid = sid * NC + cid

    def _zr(r, carry):
        for c in range(HID // 16):
            rows0[r, pl.ds(c * 16, 16)] = jnp.zeros((16,), jnp.float32)
        return carry
    lax.fori_loop(0, GROWS, _zr, 0)
    zbase = sid * ZSPLIT

    @pl.when(sid < NS - 1)
    def _():
        for z in range(ZSPLIT // GROWS):
            pltpu.sync_copy(rows0, shared.at[pl.ds(zbase + z * GROWS, GROWS)])
        zrem = ZSPLIT % GROWS
        pltpu.sync_copy(rows0.at[pl.ds(0, zrem)],
                        shared.at[pl.ds(zbase + ZSPLIT - zrem, zrem)])

    @pl.when(sid == NS - 1)
    def _():
        zlast = NPAD - (NS - 1) * ZSPLIT
        lbase = (NS - 1) * ZSPLIT
        for z in range(zlast // GROWS):
            pltpu.sync_copy(rows0, shared.at[pl.ds(lbase + z * GROWS, GROWS)])
        zrem = zlast % GROWS
        pltpu.sync_copy(rows0.at[pl.ds(0, zrem)],
                        shared.at[pl.ds(lbase + zlast - zrem, zrem)])
    plsc.subcore_barrier()

    base_row = wid * IDXROWS_PER_TILE

    def _fire(j, buf, sem):
        pltpu.async_copy(h_hbm.at[sidx.at[j]], buf, sem)

    def _drain(buf, sem):
        pltpu.make_async_copy(h_hbm.at[pl.ds(0, GROWS)], buf, sem).wait()

    def _scat(j, buf):
        pltpu.sync_copy(buf, shared.at[didx.at[j]], add=True)

    for p in range(N_PHASES):
        r0 = base_row + p * PHASE_IDXROWS
        pltpu.sync_copy(src_hbm.at[pl.ds(r0, PHASE_IDXROWS)], sidx)
        pltpu.sync_copy(dst_hbm.at[pl.ds(r0, PHASE_IDXROWS)], didx)
        _fire(0, rows0, sem0)
        _fire(1, rows1, sem1)

        def _body(k, carry):
            j0 = 2 * k
            _drain(rows0, sem0)
            _scat(j0, rows0)
            _fire(j0 + 2, rows0, sem0)
            _drain(rows1, sem1)
            _scat(j0 + 1, rows1)
            _fire(j0 + 3, rows1, sem1)
            return carry
        lax.fori_loop(0, PHASE_IDXROWS // 2 - 1, _body, 0)
        _drain(rows0, sem0)
        _scat(PHASE_IDXROWS - 2, rows0)
        _drain(rows1, sem1)
        _scat(PHASE_IDXROWS - 1, rows1)
    plsc.subcore_barrier()

    obase = sid * OSPLIT

    @pl.when(jnp.logical_and(cid == 0, sid < NS - 1))
    def _():
        pltpu.sync_copy(shared.at[pl.ds(obase, OSPLIT)],
                        out0_hbm.at[pl.ds(obase, OSPLIT)])

    @pl.when(jnp.logical_and(cid == 0, sid == NS - 1))
    def _():
        olast = N_NODES - (NS - 1) * OSPLIT
        pltpu.sync_copy(shared.at[pl.ds((NS - 1) * OSPLIT, olast)],
                        out0_hbm.at[pl.ds((NS - 1) * OSPLIT, olast)])

    @pl.when(jnp.logical_and(cid == 1, sid < NS - 1))
    def _():
        pltpu.sync_copy(shared.at[pl.ds(obase, OSPLIT)],
                        out1_hbm.at[pl.ds(obase, OSPLIT)])

    @pl.when(jnp.logical_and(cid == 1, sid == NS - 1))
    def _():
        olast = N_NODES - (NS - 1) * OSPLIT
        pltpu.sync_copy(shared.at[pl.ds((NS - 1) * OSPLIT, olast)],
                        out1_hbm.at[pl.ds((NS - 1) * OSPLIT, olast)])


_agg_kernel = functools.partial(
    pl.kernel,
    out_type=(jax.ShapeDtypeStruct((N_NODES, HID), jnp.float32),
              jax.ShapeDtypeStruct((N_NODES, HID), jnp.float32)),
    mesh=plsc.VectorSubcoreMesh(core_axis_name="c", subcore_axis_name="s",
                                num_cores=NC, num_subcores=NS),
    scratch_types=[
        pltpu.VMEM_SHARED((NPAD, HID), jnp.float32),
        pltpu.VMEM((GROWS, HID), jnp.float32),
        pltpu.VMEM((GROWS, HID), jnp.float32),
        pltpu.VMEM((PHASE_IDXROWS, 128), jnp.int32),
        pltpu.VMEM((PHASE_IDXROWS, 128), jnp.int32),
        pltpu.SemaphoreType.DMA,
        pltpu.SemaphoreType.DMA,
    ],
)(_agg_body)


def _encode_body(x_ref, w_ref, b_ref, o_ref):
    o_ref[...] = jnp.dot(x_ref[...], w_ref[...],
                         preferred_element_type=jnp.float32) + b_ref[...]


def _mlp_body(h_ref, a0_ref, a1_ref, w1_ref, b1_ref, w2_ref, b2_ref,
              g_ref, bt_ref, sc_ref, o_ref):
    h = h_ref[...]
    m = sc_ref[0, 0] * h + a0_ref[...] + a1_ref[...]
    t = jnp.dot(m, w1_ref[...], preferred_element_type=jnp.float32) + b1_ref[...]
    t = jnp.maximum(t, 0.0)
    t = jnp.dot(t, w2_ref[...], preferred_element_type=jnp.float32) + b2_ref[...]
    t = g_ref[...] * (t * INV_STD) + bt_ref[...]
    o_ref[...] = jnp.maximum(t, 0.0) + h


def _head_body(h_ref, b_ref, wp1_ref, bp1_ref, wp2_ref, bp2_ref, o_ref,
               accp_ref, accc_ref):
    i = pl.program_id(0)

    @pl.when(i == 0)
    def _():
        accp_ref[...] = jnp.zeros_like(accp_ref)
        accc_ref[...] = jnp.zeros_like(accc_ref)

    bb = b_ref[...]
    gi = lax.broadcasted_iota(jnp.int32, (1, N_GRAPHS), 1)
    onehot = (bb == gi).astype(jnp.float32)
    hb = h_ref[...]
    accp_ref[...] += lax.dot_general(
        onehot, hb, (((0,), (0,)), ((), ())),
        preferred_element_type=jnp.float32)
    ones = jnp.ones((bb.shape[0], 1), jnp.float32)
    accc_ref[...] += lax.dot_general(
        onehot, ones, (((0,), (0,)), ((), ())),
        preferred_element_type=jnp.float32)

    @pl.when(i == pl.num_programs(0) - 1)
    def _():
        pooled = accp_ref[...] / jnp.maximum(accc_ref[...], 1.0)
        t = jnp.dot(pooled, wp1_ref[...],
                    preferred_element_type=jnp.float32) + bp1_ref[...]
        t = jnp.maximum(t, 0.0)
        o_ref[...] = jnp.dot(t, wp2_ref[...],
                             preferred_element_type=jnp.float32) + bp2_ref[...]


_BLK = 2000
_HEAD_BLK = 1000


def _full_spec(shape):
    return pl.BlockSpec(shape, lambda i: (0,) * len(shape))


def _encode(x, w, b):
    return pl.pallas_call(
        _encode_body,
        grid=(N_NODES // _BLK,),
        in_specs=[
            pl.BlockSpec((_BLK, HID), lambda i: (i, 0)),
            _full_spec((HID, HID)),
            _full_spec((1, HID)),
        ],
        out_specs=pl.BlockSpec((_BLK, HID), lambda i: (i, 0)),
        out_shape=jax.ShapeDtypeStruct((N_NODES, HID), jnp.float32),
    )(x, w, b)


def _mlp(h, a0, a1, w1, b1, w2, b2, g, bt, sc):
    return pl.pallas_call(
        _mlp_body,
        grid=(N_NODES // _BLK,),
        in_specs=[
            pl.BlockSpec((_BLK, HID), lambda i: (i, 0)),
            pl.BlockSpec((_BLK, HID), lambda i: (i, 0)),
            pl.BlockSpec((_BLK, HID), lambda i: (i, 0)),
            _full_spec((HID, HID)),
            _full_spec((1, HID)),
            _full_spec((HID, HID)),
            _full_spec((1, HID)),
            _full_spec((1, HID)),
            _full_spec((1, HID)),
            pl.BlockSpec(memory_space=pltpu.SMEM),
        ],
        out_specs=pl.BlockSpec((_BLK, HID), lambda i: (i, 0)),
        out_shape=jax.ShapeDtypeStruct((N_NODES, HID), jnp.float32),
    )(h, a0, a1, w1, b1, w2, b2, g, bt, sc)


def _head(h, batchf, wp1, bp1, wp2, bp2):
    return pl.pallas_call(
        _head_body,
        grid=(N_NODES // _HEAD_BLK,),
        in_specs=[
            pl.BlockSpec((_HEAD_BLK, HID), lambda i: (i, 0)),
            pl.BlockSpec((_HEAD_BLK, 1), lambda i: (i, 0)),
            _full_spec((HID, HID)),
            _full_spec((1, HID)),
            _full_spec((HID, EMB_DIM)),
            _full_spec((1, EMB_DIM)),
        ],
        out_specs=_full_spec((N_GRAPHS, EMB_DIM)),
        out_shape=jax.ShapeDtypeStruct((N_GRAPHS, EMB_DIM), jnp.float32),
        scratch_shapes=[
            pltpu.VMEM((N_GRAPHS, HID), jnp.float32),
            pltpu.VMEM((N_GRAPHS, 1), jnp.float32),
        ],
        compiler_params=pltpu.CompilerParams(
            dimension_semantics=("arbitrary",)),
    )(h, batchf, wp1, bp1, wp2, bp2)


def kernel(x, edge_index, batch, W_enc, b_enc, eps, W1, b1, W2, b2,
           gamma, beta, Wp1, bp1, Wp2, bp2):
    src = edge_index[0]
    dst = edge_index[1]
    pad = E_PAD - N_EDGES
    src2 = jnp.concatenate(
        [src, jnp.zeros((pad,), jnp.int32)]).reshape(E_PAD // 128, 128)
    dst2 = jnp.concatenate(
        [dst, jnp.full((pad,), N_NODES, jnp.int32)]).reshape(E_PAD // 128, 128)
    batchf = batch.reshape(N_NODES, 1)

    h = _encode(x, W_enc, b_enc.reshape(1, HID))
    for i in range(N_LAYERS):
        a0, a1 = _agg_kernel(h, src2, dst2)
        h = _mlp(h, a0, a1,
                 W1[i], b1[i].reshape(1, HID),
                 W2[i], b2[i].reshape(1, HID),
                 gamma[i].reshape(1, HID), beta[i].reshape(1, HID),
                 (1.0 + eps[i]).reshape(1, 1))
    return _head(h, batchf, Wp1, bp1.reshape(1, HID), Wp2, bp2.reshape(1, EMB_DIM))
